# baseline (speedup 1.0000x reference)
import functools

import jax
import jax.numpy as jnp
from jax import lax
from jax.experimental import pallas as pl
from jax.experimental.pallas import tpu as pltpu
from jax.experimental.pallas import tpu_sc as plsc

_N = 10000
_E = 320000
_D = 128
_EPS = 1e-5

_NC = 2
_NS = 16
_NW = _NC * _NS
_EPW = _E // _NW
_CH = 125
_NCH = _EPW // _CH
_SBC = 40
_NSB = _NCH // _SBC
_RPT = 624
_RPT_LAST = _N - _RPT * (_NS - 1)
_DGW = 16

_TCB = 1000


def _tile_rows_copy(s, copy):
    @pl.when(s < _NS - 1)
    def _():
        copy(pl.multiple_of(s * _RPT, 8), _RPT)

    @pl.when(s == _NS - 1)
    def _():
        copy(pl.multiple_of(s * _RPT, 8), _RPT_LAST)


def _seg_sum_body(y, srcT, dstT, z128, agg_out, acc, src_sb, dst_sb,
                  buf_a, buf_b, gsem_a, gsem_b, ssem_a, ssem_b):
    c = lax.axis_index("c")
    s = lax.axis_index("s")
    wid = s * _NC + c
    _tile_rows_copy(s, lambda o, n: pltpu.sync_copy(z128.at[pl.ds(o, n)],
                                                    acc.at[pl.ds(o, n)]))
    plsc.subcore_barrier()

    def wait_gather(buf, sem):
        pltpu.make_async_copy(y.at[src_sb.at[0]], buf, sem).wait()

    def wait_scatter(j, buf, sem):
        pltpu.make_async_copy(buf, acc.at[dst_sb.at[j]], sem).wait()

    for sb in range(_NSB):
        pltpu.sync_copy(srcT.at[wid, pl.ds(sb * _SBC, _SBC)], src_sb)
        pltpu.sync_copy(dstT.at[wid, pl.ds(sb * _SBC, _SBC)], dst_sb)
        pltpu.async_copy(y.at[src_sb.at[0]], buf_a, gsem_a)
        pltpu.async_copy(y.at[src_sb.at[1]], buf_b, gsem_b)

        def pair(i, carry):
            j = 2 * i
            wait_gather(buf_a, gsem_a)
            pltpu.async_copy(buf_a, acc.at[dst_sb.at[j]], ssem_a, add=True)
            wait_gather(buf_b, gsem_b)
            pltpu.async_copy(buf_b, acc.at[dst_sb.at[j + 1]], ssem_b,
                             add=True)

            @pl.when(i < _SBC // 2 - 1)
            def _():
                wait_scatter(j, buf_a, ssem_a)
                pltpu.async_copy(y.at[src_sb.at[j + 2]], buf_a, gsem_a)
                wait_scatter(j + 1, buf_b, ssem_b)
                pltpu.async_copy(y.at[src_sb.at[j + 3]], buf_b, gsem_b)

            @pl.when(i == _SBC // 2 - 1)
            def _():
                wait_scatter(j, buf_a, ssem_a)
                wait_scatter(j + 1, buf_b, ssem_b)

            return carry

        lax.fori_loop(0, _SBC // 2, pair, 0)

    plsc.subcore_barrier()
    _tile_rows_copy(s, lambda o, n: pltpu.sync_copy(
        acc.at[pl.ds(o, n)], agg_out.at[c, pl.ds(o, n)]))


def _make_seg_sum():
    mesh = plsc.VectorSubcoreMesh(core_axis_name="c", subcore_axis_name="s")
    return pl.kernel(
        _seg_sum_body,
        out_type=jax.ShapeDtypeStruct((_NC, _N, _D), jnp.float32),
        mesh=mesh,
        scratch_types=[
            pltpu.VMEM_SHARED((_N, _D), jnp.float32),
            pltpu.VMEM((_SBC, _CH), jnp.int32),
            pltpu.VMEM((_SBC, _CH), jnp.int32),
            pltpu.VMEM((_CH, _D), jnp.float32),
            pltpu.VMEM((_CH, _D), jnp.float32),
            pltpu.SemaphoreType.DMA,
            pltpu.SemaphoreType.DMA,
            pltpu.SemaphoreType.DMA,
            pltpu.SemaphoreType.DMA,
        ],
    )


def _deg_body(dstT, z128, ones_h, deg_out, dega, dst_v, ones_v, sem):
    c = lax.axis_index("c")
    s = lax.axis_index("s")
    wid = s * _NC + c
    _tile_rows_copy(s, lambda o, n: pltpu.sync_copy(z128.at[pl.ds(o, n)],
                                                    dega.at[pl.ds(o, n)]))
    pltpu.sync_copy(ones_h, ones_v)
    pltpu.sync_copy(dstT.at[wid], dst_v)
    plsc.subcore_barrier()

    def chunk(j, carry):
        pltpu.async_copy(ones_v, dega.at[dst_v.at[j]], sem, add=True)

        @pl.when(j >= 4)
        def _():
            pltpu.make_async_copy(ones_v, dega.at[dst_v.at[0]], sem).wait()

        return carry

    lax.fori_loop(0, _NCH, chunk, 0)
    for _ in range(4):
        pltpu.make_async_copy(ones_v, dega.at[dst_v.at[0]], sem).wait()
    plsc.subcore_barrier()
    _tile_rows_copy(s, lambda o, n: pltpu.sync_copy(
        dega.at[pl.ds(o, n)], deg_out.at[c, pl.ds(o, n)]))


def _make_deg():
    mesh = plsc.VectorSubcoreMesh(core_axis_name="c", subcore_axis_name="s")
    return pl.kernel(
        _deg_body,
        out_type=jax.ShapeDtypeStruct((_NC, _N, _D), jnp.float32),
        mesh=mesh,
        scratch_types=[
            pltpu.VMEM_SHARED((_N, _D), jnp.float32),
            pltpu.VMEM((_NCH, _CH), jnp.int32),
            pltpu.VMEM((_CH, _D), jnp.float32),
            pltpu.SemaphoreType.DMA,
        ],
    )


def _mm_body(x_ref, w_ref, b_ref, o_ref):
    o_ref[...] = (
        jnp.dot(x_ref[...], w_ref[...], preferred_element_type=jnp.float32)
        + b_ref[...]
    )


def _mm(x, w, b):
    n_out = w.shape[1]
    return pl.pallas_call(
        _mm_body,
        grid=(_N // _TCB,),
        in_specs=[
            pl.BlockSpec((_TCB, _D), lambda i: (i, 0)),
            pl.BlockSpec((_D, n_out), lambda i: (0, 0)),
            pl.BlockSpec((1, n_out), lambda i: (0, 0)),
        ],
        out_specs=pl.BlockSpec((_TCB, n_out), lambda i: (i, 0)),
        out_shape=jax.ShapeDtypeStruct((_N, n_out), jnp.float32),
    )(x, w, b)


def _combine_mm_body(agg_ref, degp_ref, yr_ref, g_ref, be_ref, w_ref, b_ref,
                     o_ref):
    inv = 1.0 / jnp.maximum(degp_ref[0] + degp_ref[1], 1.0)
    h = (agg_ref[0] + agg_ref[1]) * inv + yr_ref[...]
    bn_scale = g_ref[...] * (1.0 / (1.0 + _EPS) ** 0.5)
    h = jnp.maximum(h * bn_scale + be_ref[...], 0.0)
    o_ref[...] = (
        jnp.dot(h, w_ref[...], preferred_element_type=jnp.float32) + b_ref[...]
    )


def _combine_mm(agg, degp, yr, g, be, w, b):
    n_out = w.shape[1]
    return pl.pallas_call(
        _combine_mm_body,
        grid=(_N // _TCB,),
        in_specs=[
            pl.BlockSpec((_NC, _TCB, _D), lambda i: (0, i, 0)),
            pl.BlockSpec((_NC, _TCB, _D), lambda i: (0, i, 0)),
            pl.BlockSpec((_TCB, _D), lambda i: (i, 0)),
            pl.BlockSpec((1, _D), lambda i: (0, 0)),
            pl.BlockSpec((1, _D), lambda i: (0, 0)),
            pl.BlockSpec((_D, n_out), lambda i: (0, 0)),
            pl.BlockSpec((1, n_out), lambda i: (0, 0)),
        ],
        out_specs=pl.BlockSpec((_TCB, n_out), lambda i: (i, 0)),
        out_shape=jax.ShapeDtypeStruct((_N, n_out), jnp.float32),
    )(agg, degp, yr, g, be, w, b)


def _final_body(agg_ref, degp_ref, yr_ref, o_ref):
    inv = 1.0 / jnp.maximum(degp_ref[0] + degp_ref[1], 1.0)
    o = (agg_ref[0] + agg_ref[1]) * inv + yr_ref[...]
    m = jnp.max(o, axis=-1, keepdims=True)
    lse = jnp.log(jnp.sum(jnp.exp(o - m), axis=-1, keepdims=True)) + m
    o_ref[...] = o - lse


def _final(agg, degp, yr):
    return pl.pallas_call(
        _final_body,
        grid=(_N // _TCB,),
        in_specs=[
            pl.BlockSpec((_NC, _TCB, _D), lambda i: (0, i, 0)),
            pl.BlockSpec((_NC, _TCB, _D), lambda i: (0, i, 0)),
            pl.BlockSpec((_TCB, _D), lambda i: (i, 0)),
        ],
        out_specs=pl.BlockSpec((_TCB, _D), lambda i: (i, 0)),
        out_shape=jax.ShapeDtypeStruct((_N, _D), jnp.float32),
    )(agg, degp, yr)


def kernel(x, edge_index, W1l, b1l, W1r, b1r, g1, be1, W2l, b2l, W2r, b2r):
    srcT = edge_index[0].reshape(_NW, _NCH, _CH)
    dstT = edge_index[1].reshape(_NW, _NCH, _CH)
    z128 = jnp.zeros((_N, _D), jnp.float32)
    ones_h = jnp.ones((_CH, _D), jnp.float32)

    Wc1 = jnp.concatenate([W1l, W1r], axis=1)
    bc1 = jnp.concatenate([jnp.zeros_like(b1l), b1l + b1r])[None, :]
    z1 = _mm(x, Wc1, bc1)
    y1l = z1[:, :_D]
    y1r = z1[:, _D:]

    degp = _make_deg()(dstT, z128, ones_h)
    agg1 = _make_seg_sum()(y1l, srcT, dstT, z128)

    Wc2 = jnp.concatenate([W2l, W2r], axis=1)
    bc2 = jnp.concatenate([jnp.zeros_like(b2l), b2l + b2r])[None, :]
    z2 = _combine_mm(agg1, degp, y1r, g1[None, :], be1[None, :], Wc2, bc2)
    y2l = z2[:, :_D]
    y2r = z2[:, _D:]

    agg2 = _make_seg_sum()(y2l, srcT, dstT, z128)

    return _final(agg2, degp, y2r)


# speedup vs baseline: 1.1873x; 1.1873x over previous
import functools

import jax
import jax.numpy as jnp
from jax import lax
from jax.experimental import pallas as pl
from jax.experimental.pallas import tpu as pltpu
from jax.experimental.pallas import tpu_sc as plsc

_N = 10000
_E = 320000
_D = 128
_EPS = 1e-5

_NC = 2
_NS = 16
_NW = _NC * _NS
_EPW = _E // _NW
_CH = 125
_NCH = _EPW // _CH
_SBC = 40
_NSB = _NCH // _SBC
_RPT = 624
_RPT_LAST = _N - _RPT * (_NS - 1)
_DGW = 16

_TCB = 1000


def _tile_rows_copy(s, copy):
    @pl.when(s < _NS - 1)
    def _():
        copy(pl.multiple_of(s * _RPT, 8), _RPT)

    @pl.when(s == _NS - 1)
    def _():
        copy(pl.multiple_of(s * _RPT, 8), _RPT_LAST)


def _seg_sum_body(y, srcT, dstT, z128, agg_out, acc, src_sb, dst_sb,
                  buf_a, buf_b, gsem_a, gsem_b, ssem_a, ssem_b):
    c = lax.axis_index("c")
    s = lax.axis_index("s")
    wid = s * _NC + c
    _tile_rows_copy(s, lambda o, n: pltpu.sync_copy(z128.at[pl.ds(o, n)],
                                                    acc.at[pl.ds(o, n)]))
    plsc.subcore_barrier()

    def wait_gather(buf, sem):
        pltpu.make_async_copy(y.at[src_sb.at[0]], buf, sem).wait()

    def wait_scatter(j, buf, sem):
        pltpu.make_async_copy(buf, acc.at[dst_sb.at[j]], sem).wait()

    for sb in range(_NSB):
        pltpu.sync_copy(srcT.at[wid, pl.ds(sb * _SBC, _SBC)], src_sb)
        pltpu.sync_copy(dstT.at[wid, pl.ds(sb * _SBC, _SBC)], dst_sb)
        pltpu.async_copy(y.at[src_sb.at[0]], buf_a, gsem_a)
        pltpu.async_copy(y.at[src_sb.at[1]], buf_b, gsem_b)

        def pair(i, carry):
            j = 2 * i
            wait_gather(buf_a, gsem_a)
            pltpu.sync_copy(buf_a, acc.at[dst_sb.at[j]], add=True)

            @pl.when(i < _SBC // 2 - 1)
            def _():
                pltpu.async_copy(y.at[src_sb.at[j + 2]], buf_a, gsem_a)

            wait_gather(buf_b, gsem_b)
            pltpu.sync_copy(buf_b, acc.at[dst_sb.at[j + 1]], add=True)

            @pl.when(i < _SBC // 2 - 1)
            def _():
                pltpu.async_copy(y.at[src_sb.at[j + 3]], buf_b, gsem_b)

            return carry

        lax.fori_loop(0, _SBC // 2, pair, 0)

    plsc.subcore_barrier()
    _tile_rows_copy(s, lambda o, n: pltpu.sync_copy(
        acc.at[pl.ds(o, n)], agg_out.at[c, pl.ds(o, n)]))


def _make_seg_sum():
    mesh = plsc.VectorSubcoreMesh(core_axis_name="c", subcore_axis_name="s")
    return pl.kernel(
        _seg_sum_body,
        out_type=jax.ShapeDtypeStruct((_NC, _N, _D), jnp.float32),
        mesh=mesh,
        scratch_types=[
            pltpu.VMEM_SHARED((_N, _D), jnp.float32),
            pltpu.VMEM((_SBC, _CH), jnp.int32),
            pltpu.VMEM((_SBC, _CH), jnp.int32),
            pltpu.VMEM((_CH, _D), jnp.float32),
            pltpu.VMEM((_CH, _D), jnp.float32),
            pltpu.SemaphoreType.DMA,
            pltpu.SemaphoreType.DMA,
            pltpu.SemaphoreType.DMA,
            pltpu.SemaphoreType.DMA,
        ],
    )


def _deg_body(dstT, z128, ones_h, deg_out, dega, dst_v, ones_v, sem):
    c = lax.axis_index("c")
    s = lax.axis_index("s")
    wid = s * _NC + c
    _tile_rows_copy(s, lambda o, n: pltpu.sync_copy(z128.at[pl.ds(o, n)],
                                                    dega.at[pl.ds(o, n)]))
    pltpu.sync_copy(ones_h, ones_v)
    pltpu.sync_copy(dstT.at[wid], dst_v)
    plsc.subcore_barrier()

    def chunk(j, carry):
        pltpu.async_copy(ones_v, dega.at[dst_v.at[j]], sem, add=True)

        @pl.when(j >= 4)
        def _():
            pltpu.make_async_copy(ones_v, dega.at[dst_v.at[0]], sem).wait()

        return carry

    lax.fori_loop(0, _NCH, chunk, 0)
    for _ in range(4):
        pltpu.make_async_copy(ones_v, dega.at[dst_v.at[0]], sem).wait()
    plsc.subcore_barrier()
    _tile_rows_copy(s, lambda o, n: pltpu.sync_copy(
        dega.at[pl.ds(o, n)], deg_out.at[c, pl.ds(o, n)]))


def _make_deg():
    mesh = plsc.VectorSubcoreMesh(core_axis_name="c", subcore_axis_name="s")
    return pl.kernel(
        _deg_body,
        out_type=jax.ShapeDtypeStruct((_NC, _N, _D), jnp.float32),
        mesh=mesh,
        scratch_types=[
            pltpu.VMEM_SHARED((_N, _D), jnp.float32),
            pltpu.VMEM((_NCH, _CH), jnp.int32),
            pltpu.VMEM((_CH, _D), jnp.float32),
            pltpu.SemaphoreType.DMA,
        ],
    )


def _mm_body(x_ref, w_ref, b_ref, o_ref):
    o_ref[...] = (
        jnp.dot(x_ref[...], w_ref[...], preferred_element_type=jnp.float32)
        + b_ref[...]
    )


def _mm(x, w, b):
    n_out = w.shape[1]
    return pl.pallas_call(
        _mm_body,
        grid=(_N // _TCB,),
        in_specs=[
            pl.BlockSpec((_TCB, _D), lambda i: (i, 0)),
            pl.BlockSpec((_D, n_out), lambda i: (0, 0)),
            pl.BlockSpec((1, n_out), lambda i: (0, 0)),
        ],
        out_specs=pl.BlockSpec((_TCB, n_out), lambda i: (i, 0)),
        out_shape=jax.ShapeDtypeStruct((_N, n_out), jnp.float32),
    )(x, w, b)


def _combine_mm_body(agg_ref, degp_ref, yr_ref, g_ref, be_ref, w_ref, b_ref,
                     o_ref):
    inv = 1.0 / jnp.maximum(degp_ref[0] + degp_ref[1], 1.0)
    h = (agg_ref[0] + agg_ref[1]) * inv + yr_ref[...]
    bn_scale = g_ref[...] * (1.0 / (1.0 + _EPS) ** 0.5)
    h = jnp.maximum(h * bn_scale + be_ref[...], 0.0)
    o_ref[...] = (
        jnp.dot(h, w_ref[...], preferred_element_type=jnp.float32) + b_ref[...]
    )


def _combine_mm(agg, degp, yr, g, be, w, b):
    n_out = w.shape[1]
    return pl.pallas_call(
        _combine_mm_body,
        grid=(_N // _TCB,),
        in_specs=[
            pl.BlockSpec((_NC, _TCB, _D), lambda i: (0, i, 0)),
            pl.BlockSpec((_NC, _TCB, _D), lambda i: (0, i, 0)),
            pl.BlockSpec((_TCB, _D), lambda i: (i, 0)),
            pl.BlockSpec((1, _D), lambda i: (0, 0)),
            pl.BlockSpec((1, _D), lambda i: (0, 0)),
            pl.BlockSpec((_D, n_out), lambda i: (0, 0)),
            pl.BlockSpec((1, n_out), lambda i: (0, 0)),
        ],
        out_specs=pl.BlockSpec((_TCB, n_out), lambda i: (i, 0)),
        out_shape=jax.ShapeDtypeStruct((_N, n_out), jnp.float32),
    )(agg, degp, yr, g, be, w, b)


def _final_body(agg_ref, degp_ref, yr_ref, o_ref):
    inv = 1.0 / jnp.maximum(degp_ref[0] + degp_ref[1], 1.0)
    o = (agg_ref[0] + agg_ref[1]) * inv + yr_ref[...]
    m = jnp.max(o, axis=-1, keepdims=True)
    lse = jnp.log(jnp.sum(jnp.exp(o - m), axis=-1, keepdims=True)) + m
    o_ref[...] = o - lse


def _final(agg, degp, yr):
    return pl.pallas_call(
        _final_body,
        grid=(_N // _TCB,),
        in_specs=[
            pl.BlockSpec((_NC, _TCB, _D), lambda i: (0, i, 0)),
            pl.BlockSpec((_NC, _TCB, _D), lambda i: (0, i, 0)),
            pl.BlockSpec((_TCB, _D), lambda i: (i, 0)),
        ],
        out_specs=pl.BlockSpec((_TCB, _D), lambda i: (i, 0)),
        out_shape=jax.ShapeDtypeStruct((_N, _D), jnp.float32),
    )(agg, degp, yr)


def kernel(x, edge_index, W1l, b1l, W1r, b1r, g1, be1, W2l, b2l, W2r, b2r):
    srcT = edge_index[0].reshape(_NW, _NCH, _CH)
    dstT = edge_index[1].reshape(_NW, _NCH, _CH)
    z128 = jnp.zeros((_N, _D), jnp.float32)
    ones_h = jnp.ones((_CH, _D), jnp.float32)

    Wc1 = jnp.concatenate([W1l, W1r], axis=1)
    bc1 = jnp.concatenate([jnp.zeros_like(b1l), b1l + b1r])[None, :]
    z1 = _mm(x, Wc1, bc1)
    y1l = z1[:, :_D]
    y1r = z1[:, _D:]

    degp = _make_deg()(dstT, z128, ones_h)
    agg1 = _make_seg_sum()(y1l, srcT, dstT, z128)

    Wc2 = jnp.concatenate([W2l, W2r], axis=1)
    bc2 = jnp.concatenate([jnp.zeros_like(b2l), b2l + b2r])[None, :]
    z2 = _combine_mm(agg1, degp, y1r, g1[None, :], be1[None, :], Wc2, bc2)
    y2l = z2[:, :_D]
    y2r = z2[:, _D:]

    agg2 = _make_seg_sum()(y2l, srcT, dstT, z128)

    return _final(agg2, degp, y2r)


# speedup vs baseline: 1.2049x; 1.0148x over previous
import functools

import jax
import jax.numpy as jnp
from jax import lax
from jax.experimental import pallas as pl
from jax.experimental.pallas import tpu as pltpu
from jax.experimental.pallas import tpu_sc as plsc

_N = 10000
_E = 320000
_D = 128
_EPS = 1e-5

_NC = 2
_NS = 16
_NW = _NC * _NS
_EPW = _E // _NW
_CH = 125
_NCH = _EPW // _CH
_SBC = 40
_NSB = _NCH // _SBC

_RCH = 100
_RNCH = _EPW // _RCH
_RSBC = 20
_RNSB = _RNCH // _RSBC
_NBUF = 3
_RPT = 624
_RPT_LAST = _N - _RPT * (_NS - 1)
_DGW = 16

_TCB = 1000


def _tile_rows_copy(s, copy):
    @pl.when(s < _NS - 1)
    def _():
        copy(pl.multiple_of(s * _RPT, 8), _RPT)

    @pl.when(s == _NS - 1)
    def _():
        copy(pl.multiple_of(s * _RPT, 8), _RPT_LAST)


def _seg_sum_body(y, srcT, dstT, z128, agg_out, acc, src_sb, dst_sb,
                  buf0, buf1, buf2, sem0, sem1, sem2):
    c = lax.axis_index("c")
    s = lax.axis_index("s")
    wid = s * _NC + c
    bufs = (buf0, buf1, buf2)
    sems = (sem0, sem1, sem2)
    _tile_rows_copy(s, lambda o, n: pltpu.sync_copy(z128.at[pl.ds(o, n)],
                                                    acc.at[pl.ds(o, n)]))
    plsc.subcore_barrier()

    def superblock(sb, carry):
        pltpu.sync_copy(srcT.at[wid, sb], src_sb)
        pltpu.sync_copy(dstT.at[wid, sb], dst_sb)
        for k in range(_NBUF):
            pltpu.async_copy(y.at[src_sb.at[k]], bufs[k], sems[k])
        for k in range(_RSBC):
            b = k % _NBUF
            pltpu.make_async_copy(y.at[src_sb.at[k]], bufs[b],
                                  sems[b]).wait()
            pltpu.sync_copy(bufs[b], acc.at[dst_sb.at[k]], add=True)
            if k + _NBUF < _RSBC:
                pltpu.async_copy(y.at[src_sb.at[k + _NBUF]], bufs[b],
                                 sems[b])
        return carry

    lax.fori_loop(0, _RNSB, superblock, 0)

    plsc.subcore_barrier()
    _tile_rows_copy(s, lambda o, n: pltpu.sync_copy(
        acc.at[pl.ds(o, n)], agg_out.at[c, pl.ds(o, n)]))


def _make_seg_sum():
    mesh = plsc.VectorSubcoreMesh(core_axis_name="c", subcore_axis_name="s")
    return pl.kernel(
        _seg_sum_body,
        out_type=jax.ShapeDtypeStruct((_NC, _N, _D), jnp.float32),
        mesh=mesh,
        scratch_types=[
            pltpu.VMEM_SHARED((_N, _D), jnp.float32),
            pltpu.VMEM((_RSBC, _RCH), jnp.int32),
            pltpu.VMEM((_RSBC, _RCH), jnp.int32),
            pltpu.VMEM((_RCH, _D), jnp.float32),
            pltpu.VMEM((_RCH, _D), jnp.float32),
            pltpu.VMEM((_RCH, _D), jnp.float32),
            pltpu.SemaphoreType.DMA,
            pltpu.SemaphoreType.DMA,
            pltpu.SemaphoreType.DMA,
        ],
    )


def _deg_body(dstT, z128, ones_h, deg_out, dega, dst_v, ones_v, sem):
    c = lax.axis_index("c")
    s = lax.axis_index("s")
    wid = s * _NC + c
    _tile_rows_copy(s, lambda o, n: pltpu.sync_copy(z128.at[pl.ds(o, n)],
                                                    dega.at[pl.ds(o, n)]))
    pltpu.sync_copy(ones_h, ones_v)
    pltpu.sync_copy(dstT.at[wid], dst_v)
    plsc.subcore_barrier()

    def chunk(j, carry):
        pltpu.async_copy(ones_v, dega.at[dst_v.at[j]], sem, add=True)

        @pl.when(j >= 4)
        def _():
            pltpu.make_async_copy(ones_v, dega.at[dst_v.at[0]], sem).wait()

        return carry

    lax.fori_loop(0, _NCH, chunk, 0)
    for _ in range(4):
        pltpu.make_async_copy(ones_v, dega.at[dst_v.at[0]], sem).wait()
    plsc.subcore_barrier()
    _tile_rows_copy(s, lambda o, n: pltpu.sync_copy(
        dega.at[pl.ds(o, n)], deg_out.at[c, pl.ds(o, n)]))


def _make_deg():
    mesh = plsc.VectorSubcoreMesh(core_axis_name="c", subcore_axis_name="s")
    return pl.kernel(
        _deg_body,
        out_type=jax.ShapeDtypeStruct((_NC, _N, _D), jnp.float32),
        mesh=mesh,
        scratch_types=[
            pltpu.VMEM_SHARED((_N, _D), jnp.float32),
            pltpu.VMEM((_NCH, _CH), jnp.int32),
            pltpu.VMEM((_CH, _D), jnp.float32),
            pltpu.SemaphoreType.DMA,
        ],
    )


def _mm_body(x_ref, w_ref, b_ref, o_ref):
    o_ref[...] = (
        jnp.dot(x_ref[...], w_ref[...], preferred_element_type=jnp.float32)
        + b_ref[...]
    )


def _mm(x, w, b):
    n_out = w.shape[1]
    return pl.pallas_call(
        _mm_body,
        grid=(_N // _TCB,),
        in_specs=[
            pl.BlockSpec((_TCB, _D), lambda i: (i, 0)),
            pl.BlockSpec((_D, n_out), lambda i: (0, 0)),
            pl.BlockSpec((1, n_out), lambda i: (0, 0)),
        ],
        out_specs=pl.BlockSpec((_TCB, n_out), lambda i: (i, 0)),
        out_shape=jax.ShapeDtypeStruct((_N, n_out), jnp.float32),
    )(x, w, b)


def _combine_mm_body(agg_ref, degp_ref, yr_ref, g_ref, be_ref, w_ref, b_ref,
                     o_ref):
    inv = 1.0 / jnp.maximum(degp_ref[0] + degp_ref[1], 1.0)
    h = (agg_ref[0] + agg_ref[1]) * inv + yr_ref[...]
    bn_scale = g_ref[...] * (1.0 / (1.0 + _EPS) ** 0.5)
    h = jnp.maximum(h * bn_scale + be_ref[...], 0.0)
    o_ref[...] = (
        jnp.dot(h, w_ref[...], preferred_element_type=jnp.float32) + b_ref[...]
    )


def _combine_mm(agg, degp, yr, g, be, w, b):
    n_out = w.shape[1]
    return pl.pallas_call(
        _combine_mm_body,
        grid=(_N // _TCB,),
        in_specs=[
            pl.BlockSpec((_NC, _TCB, _D), lambda i: (0, i, 0)),
            pl.BlockSpec((_NC, _TCB, _D), lambda i: (0, i, 0)),
            pl.BlockSpec((_TCB, _D), lambda i: (i, 0)),
            pl.BlockSpec((1, _D), lambda i: (0, 0)),
            pl.BlockSpec((1, _D), lambda i: (0, 0)),
            pl.BlockSpec((_D, n_out), lambda i: (0, 0)),
            pl.BlockSpec((1, n_out), lambda i: (0, 0)),
        ],
        out_specs=pl.BlockSpec((_TCB, n_out), lambda i: (i, 0)),
        out_shape=jax.ShapeDtypeStruct((_N, n_out), jnp.float32),
    )(agg, degp, yr, g, be, w, b)


def _final_body(agg_ref, degp_ref, yr_ref, o_ref):
    inv = 1.0 / jnp.maximum(degp_ref[0] + degp_ref[1], 1.0)
    o = (agg_ref[0] + agg_ref[1]) * inv + yr_ref[...]
    m = jnp.max(o, axis=-1, keepdims=True)
    lse = jnp.log(jnp.sum(jnp.exp(o - m), axis=-1, keepdims=True)) + m
    o_ref[...] = o - lse


def _final(agg, degp, yr):
    return pl.pallas_call(
        _final_body,
        grid=(_N // _TCB,),
        in_specs=[
            pl.BlockSpec((_NC, _TCB, _D), lambda i: (0, i, 0)),
            pl.BlockSpec((_NC, _TCB, _D), lambda i: (0, i, 0)),
            pl.BlockSpec((_TCB, _D), lambda i: (i, 0)),
        ],
        out_specs=pl.BlockSpec((_TCB, _D), lambda i: (i, 0)),
        out_shape=jax.ShapeDtypeStruct((_N, _D), jnp.float32),
    )(agg, degp, yr)


def kernel(x, edge_index, W1l, b1l, W1r, b1r, g1, be1, W2l, b2l, W2r, b2r):
    srcT = edge_index[0].reshape(_NW, _RNSB, _RSBC, _RCH)
    dstT = edge_index[1].reshape(_NW, _RNSB, _RSBC, _RCH)
    dstTd = edge_index[1].reshape(_NW, _NCH, _CH)
    z128 = jnp.zeros((_N, _D), jnp.float32)
    ones_h = jnp.ones((_CH, _D), jnp.float32)

    Wc1 = jnp.concatenate([W1l, W1r], axis=1)
    bc1 = jnp.concatenate([jnp.zeros_like(b1l), b1l + b1r])[None, :]
    z1 = _mm(x, Wc1, bc1)
    y1l = z1[:, :_D]
    y1r = z1[:, _D:]

    degp = _make_deg()(dstTd, z128, ones_h)
    agg1 = _make_seg_sum()(y1l, srcT, dstT, z128)

    Wc2 = jnp.concatenate([W2l, W2r], axis=1)
    bc2 = jnp.concatenate([jnp.zeros_like(b2l), b2l + b2r])[None, :]
    z2 = _combine_mm(agg1, degp, y1r, g1[None, :], be1[None, :], Wc2, bc2)
    y2l = z2[:, :_D]
    y2r = z2[:, _D:]

    agg2 = _make_seg_sum()(y2l, srcT, dstT, z128)

    return _final(agg2, degp, y2r)


# speedup vs baseline: 1.2513x; 1.0385x over previous
import functools

import jax
import jax.numpy as jnp
from jax import lax
from jax.experimental import pallas as pl
from jax.experimental.pallas import tpu as pltpu
from jax.experimental.pallas import tpu_sc as plsc

_N = 10000
_E = 320000
_D = 128
_EPS = 1e-5

_NC = 2
_NS = 16
_NW = _NC * _NS
_EPW = _E // _NW
_CH = 125
_NCH = _EPW // _CH
_SBC = 40
_NSB = _NCH // _SBC

_RCH = 100
_RNCH = _EPW // _RCH
_RSBC = 20
_RNSB = _RNCH // _RSBC
_NBUF = 3
_RPT = 624
_RPT_LAST = _N - _RPT * (_NS - 1)
_DGW = 16

_TCB = 1000


def _tile_rows_copy(s, copy):
    @pl.when(s < _NS - 1)
    def _():
        copy(pl.multiple_of(s * _RPT, 8), _RPT)

    @pl.when(s == _NS - 1)
    def _():
        copy(pl.multiple_of(s * _RPT, 8), _RPT_LAST)


def _seg_sum_body(y, srcT, dstT, z128, agg_out, acc, src_sb, dst_sb,
                  buf0, buf1, buf2, sem0, sem1, sem2):
    c = lax.axis_index("c")
    s = lax.axis_index("s")
    wid = s * _NC + c
    bufs = (buf0, buf1, buf2)
    sems = (sem0, sem1, sem2)
    _tile_rows_copy(s, lambda o, n: pltpu.sync_copy(z128.at[pl.ds(o, n)],
                                                    acc.at[pl.ds(o, n)]))
    plsc.subcore_barrier()

    def superblock(sb, carry):
        pltpu.sync_copy(srcT.at[wid, sb], src_sb)
        pltpu.sync_copy(dstT.at[wid, sb], dst_sb)
        for k in range(_NBUF):
            pltpu.async_copy(y.at[src_sb.at[k]], bufs[k], sems[k])
        for k in range(_RSBC):
            b = k % _NBUF
            pltpu.make_async_copy(y.at[src_sb.at[k]], bufs[b],
                                  sems[b]).wait()
            pltpu.sync_copy(bufs[b], acc.at[dst_sb.at[k]], add=True)
            if k + _NBUF < _RSBC:
                pltpu.async_copy(y.at[src_sb.at[k + _NBUF]], bufs[b],
                                 sems[b])
        return carry

    lax.fori_loop(0, _RNSB, superblock, 0)

    plsc.subcore_barrier()
    _tile_rows_copy(s, lambda o, n: pltpu.sync_copy(
        acc.at[pl.ds(o, n)], agg_out.at[c, pl.ds(o, n)]))


def _make_seg_sum():
    mesh = plsc.VectorSubcoreMesh(core_axis_name="c", subcore_axis_name="s")
    return pl.kernel(
        _seg_sum_body,
        out_type=jax.ShapeDtypeStruct((_NC, _N, _D), jnp.float32),
        mesh=mesh,
        scratch_types=[
            pltpu.VMEM_SHARED((_N, _D), jnp.float32),
            pltpu.VMEM((_RSBC, _RCH), jnp.int32),
            pltpu.VMEM((_RSBC, _RCH), jnp.int32),
            pltpu.VMEM((_RCH, _D), jnp.float32),
            pltpu.VMEM((_RCH, _D), jnp.float32),
            pltpu.VMEM((_RCH, _D), jnp.float32),
            pltpu.SemaphoreType.DMA,
            pltpu.SemaphoreType.DMA,
            pltpu.SemaphoreType.DMA,
        ],
    )


def _deg_body(dstT, z128, ones_h, deg_out, dega, dst_v, ones_v, sem):
    c = lax.axis_index("c")
    s = lax.axis_index("s")
    wid = s * _NC + c
    _tile_rows_copy(s, lambda o, n: pltpu.sync_copy(z128.at[pl.ds(o, n)],
                                                    dega.at[pl.ds(o, n)]))
    pltpu.sync_copy(ones_h, ones_v)
    pltpu.sync_copy(dstT.at[wid], dst_v)
    plsc.subcore_barrier()

    def chunk(j, carry):
        pltpu.async_copy(ones_v, dega.at[dst_v.at[j]], sem, add=True)

        @pl.when(j >= 4)
        def _():
            pltpu.make_async_copy(ones_v, dega.at[dst_v.at[0]], sem).wait()

        return carry

    lax.fori_loop(0, _NCH, chunk, 0)
    for _ in range(4):
        pltpu.make_async_copy(ones_v, dega.at[dst_v.at[0]], sem).wait()
    plsc.subcore_barrier()
    _tile_rows_copy(s, lambda o, n: pltpu.sync_copy(
        dega.at[pl.ds(o, n)], deg_out.at[c, pl.ds(o, n)]))


def _make_deg():
    mesh = plsc.VectorSubcoreMesh(core_axis_name="c", subcore_axis_name="s")
    return pl.kernel(
        _deg_body,
        out_type=jax.ShapeDtypeStruct((_NC, _N, _D), jnp.float32),
        mesh=mesh,
        scratch_types=[
            pltpu.VMEM_SHARED((_N, _D), jnp.float32),
            pltpu.VMEM((_NCH, _CH), jnp.int32),
            pltpu.VMEM((_CH, _D), jnp.float32),
            pltpu.SemaphoreType.DMA,
        ],
    )


def _mm_body(x_ref, w_ref, b_ref, ol_ref, or_ref):
    z = (jnp.dot(x_ref[...], w_ref[...], preferred_element_type=jnp.float32)
         + b_ref[...])
    ol_ref[...] = z[:, :_D]
    or_ref[...] = z[:, _D:]


def _mm(x, w, b):
    return pl.pallas_call(
        _mm_body,
        grid=(_N // _TCB,),
        in_specs=[
            pl.BlockSpec((_TCB, _D), lambda i: (i, 0)),
            pl.BlockSpec((_D, 2 * _D), lambda i: (0, 0)),
            pl.BlockSpec((1, 2 * _D), lambda i: (0, 0)),
        ],
        out_specs=[
            pl.BlockSpec((_TCB, _D), lambda i: (i, 0)),
            pl.BlockSpec((_TCB, _D), lambda i: (i, 0)),
        ],
        out_shape=[
            jax.ShapeDtypeStruct((_N, _D), jnp.float32),
            jax.ShapeDtypeStruct((_N, _D), jnp.float32),
        ],
    )(x, w, b)


def _combine_mm_body(agg_ref, degp_ref, yr_ref, g_ref, be_ref, w_ref, b_ref,
                     ol_ref, or_ref):
    inv = 1.0 / jnp.maximum(degp_ref[0] + degp_ref[1], 1.0)
    h = (agg_ref[0] + agg_ref[1]) * inv + yr_ref[...]
    bn_scale = g_ref[...] * (1.0 / (1.0 + _EPS) ** 0.5)
    h = jnp.maximum(h * bn_scale + be_ref[...], 0.0)
    z = (jnp.dot(h, w_ref[...], preferred_element_type=jnp.float32)
         + b_ref[...])
    ol_ref[...] = z[:, :_D]
    or_ref[...] = z[:, _D:]


def _combine_mm(agg, degp, yr, g, be, w, b):
    return pl.pallas_call(
        _combine_mm_body,
        grid=(_N // _TCB,),
        in_specs=[
            pl.BlockSpec((_NC, _TCB, _D), lambda i: (0, i, 0)),
            pl.BlockSpec((_NC, _TCB, _D), lambda i: (0, i, 0)),
            pl.BlockSpec((_TCB, _D), lambda i: (i, 0)),
            pl.BlockSpec((1, _D), lambda i: (0, 0)),
            pl.BlockSpec((1, _D), lambda i: (0, 0)),
            pl.BlockSpec((_D, 2 * _D), lambda i: (0, 0)),
            pl.BlockSpec((1, 2 * _D), lambda i: (0, 0)),
        ],
        out_specs=[
            pl.BlockSpec((_TCB, _D), lambda i: (i, 0)),
            pl.BlockSpec((_TCB, _D), lambda i: (i, 0)),
        ],
        out_shape=[
            jax.ShapeDtypeStruct((_N, _D), jnp.float32),
            jax.ShapeDtypeStruct((_N, _D), jnp.float32),
        ],
    )(agg, degp, yr, g, be, w, b)


def _final_body(agg_ref, degp_ref, yr_ref, o_ref):
    inv = 1.0 / jnp.maximum(degp_ref[0] + degp_ref[1], 1.0)
    o = (agg_ref[0] + agg_ref[1]) * inv + yr_ref[...]
    m = jnp.max(o, axis=-1, keepdims=True)
    lse = jnp.log(jnp.sum(jnp.exp(o - m), axis=-1, keepdims=True)) + m
    o_ref[...] = o - lse


def _final(agg, degp, yr):
    return pl.pallas_call(
        _final_body,
        grid=(_N // _TCB,),
        in_specs=[
            pl.BlockSpec((_NC, _TCB, _D), lambda i: (0, i, 0)),
            pl.BlockSpec((_NC, _TCB, _D), lambda i: (0, i, 0)),
            pl.BlockSpec((_TCB, _D), lambda i: (i, 0)),
        ],
        out_specs=pl.BlockSpec((_TCB, _D), lambda i: (i, 0)),
        out_shape=jax.ShapeDtypeStruct((_N, _D), jnp.float32),
    )(agg, degp, yr)


def kernel(x, edge_index, W1l, b1l, W1r, b1r, g1, be1, W2l, b2l, W2r, b2r):
    srcT = edge_index[0].reshape(_NW, _RNSB, _RSBC, _RCH)
    dstT = edge_index[1].reshape(_NW, _RNSB, _RSBC, _RCH)
    dstTd = edge_index[1].reshape(_NW, _NCH, _CH)
    z128 = jnp.zeros((_N, _D), jnp.float32)
    ones_h = jnp.ones((_CH, _D), jnp.float32)

    Wc1 = jnp.concatenate([W1l, W1r], axis=1)
    bc1 = jnp.concatenate([jnp.zeros_like(b1l), b1l + b1r])[None, :]
    y1l, y1r = _mm(x, Wc1, bc1)

    degp = _make_deg()(dstTd, z128, ones_h)
    agg1 = _make_seg_sum()(y1l, srcT, dstT, z128)

    Wc2 = jnp.concatenate([W2l, W2r], axis=1)
    bc2 = jnp.concatenate([jnp.zeros_like(b2l), b2l + b2r])[None, :]
    y2l, y2r = _combine_mm(agg1, degp, y1r, g1[None, :], be1[None, :],
                           Wc2, bc2)

    agg2 = _make_seg_sum()(y2l, srcT, dstT, z128)

    return _final(agg2, degp, y2r)


# speedup vs baseline: 1.2650x; 1.0109x over previous
import functools

import jax
import jax.numpy as jnp
from jax import lax
from jax.experimental import pallas as pl
from jax.experimental.pallas import tpu as pltpu
from jax.experimental.pallas import tpu_sc as plsc

_N = 10000
_E = 320000
_D = 128
_EPS = 1e-5

_NC = 2
_NS = 16
_NW = _NC * _NS
_EPW = _E // _NW
_CH = 125
_NCH = _EPW // _CH
_SBC = 40
_NSB = _NCH // _SBC

_RCH = 100
_RNCH = _EPW // _RCH
_RSBC = 20
_RNSB = _RNCH // _RSBC
_NBUF = 3
_RPT = 624
_RPT_LAST = _N - _RPT * (_NS - 1)
_DGW = 16

_TCB = 1000


def _tile_rows_copy(s, copy):
    @pl.when(s < _NS - 1)
    def _():
        copy(pl.multiple_of(s * _RPT, 8), _RPT)

    @pl.when(s == _NS - 1)
    def _():
        copy(pl.multiple_of(s * _RPT, 8), _RPT_LAST)


def _seg_sum_body(with_deg, *refs):
    if with_deg:
        (y, srcT, dstT, z128, ones_h, deg_out, agg_out,
         acc, src_sb, dst_sb, buf0, buf1, buf2, sem0, sem1, sem2) = refs
    else:
        (y, srcT, dstT, z128, agg_out,
         acc, src_sb, dst_sb, buf0, buf1, buf2, sem0, sem1, sem2) = refs
    c = lax.axis_index("c")
    s = lax.axis_index("s")
    wid = s * _NC + c
    bufs = (buf0, buf1, buf2)
    sems = (sem0, sem1, sem2)
    _tile_rows_copy(s, lambda o, n: pltpu.sync_copy(z128.at[pl.ds(o, n)],
                                                    acc.at[pl.ds(o, n)]))

    if with_deg:
        pltpu.sync_copy(ones_h, buf0)
        plsc.subcore_barrier()

        def deg_superblock(sb, carry):
            pltpu.sync_copy(dstT.at[wid, sb], dst_sb)
            for k in range(_RSBC):
                pltpu.async_copy(buf0, acc.at[dst_sb.at[k]], sem0, add=True)
                if k >= 4:
                    pltpu.make_async_copy(buf0, acc.at[dst_sb.at[0]],
                                          sem0).wait()
            for _ in range(4):
                pltpu.make_async_copy(buf0, acc.at[dst_sb.at[0]],
                                      sem0).wait()
            return carry

        lax.fori_loop(0, _RNSB, deg_superblock, 0)
        plsc.subcore_barrier()
        _tile_rows_copy(s, lambda o, n: pltpu.sync_copy(
            acc.at[pl.ds(o, n)], deg_out.at[c, pl.ds(o, n)]))
        _tile_rows_copy(s, lambda o, n: pltpu.sync_copy(
            z128.at[pl.ds(o, n)], acc.at[pl.ds(o, n)]))

    plsc.subcore_barrier()

    def superblock(sb, carry):
        pltpu.sync_copy(srcT.at[wid, sb], src_sb)
        pltpu.sync_copy(dstT.at[wid, sb], dst_sb)
        for k in range(_NBUF):
            pltpu.async_copy(y.at[src_sb.at[k]], bufs[k], sems[k])
        for k in range(_RSBC):
            b = k % _NBUF
            pltpu.make_async_copy(y.at[src_sb.at[k]], bufs[b],
                                  sems[b]).wait()
            pltpu.sync_copy(bufs[b], acc.at[dst_sb.at[k]], add=True)
            if k + _NBUF < _RSBC:
                pltpu.async_copy(y.at[src_sb.at[k + _NBUF]], bufs[b],
                                 sems[b])
        return carry

    lax.fori_loop(0, _RNSB, superblock, 0)

    plsc.subcore_barrier()
    _tile_rows_copy(s, lambda o, n: pltpu.sync_copy(
        acc.at[pl.ds(o, n)], agg_out.at[c, pl.ds(o, n)]))


def _make_seg_sum(with_deg):
    mesh = plsc.VectorSubcoreMesh(core_axis_name="c", subcore_axis_name="s")
    out = jax.ShapeDtypeStruct((_NC, _N, _D), jnp.float32)
    return pl.kernel(
        functools.partial(_seg_sum_body, with_deg),
        out_type=(out, out) if with_deg else out,
        mesh=mesh,
        scratch_types=[
            pltpu.VMEM_SHARED((_N, _D), jnp.float32),
            pltpu.VMEM((_RSBC, _RCH), jnp.int32),
            pltpu.VMEM((_RSBC, _RCH), jnp.int32),
            pltpu.VMEM((_RCH, _D), jnp.float32),
            pltpu.VMEM((_RCH, _D), jnp.float32),
            pltpu.VMEM((_RCH, _D), jnp.float32),
            pltpu.SemaphoreType.DMA,
            pltpu.SemaphoreType.DMA,
            pltpu.SemaphoreType.DMA,
        ],
    )


def _mm_body(x_ref, w_ref, b_ref, ol_ref, or_ref):
    z = (jnp.dot(x_ref[...], w_ref[...], preferred_element_type=jnp.float32)
         + b_ref[...])
    ol_ref[...] = z[:, :_D]
    or_ref[...] = z[:, _D:]


def _mm(x, w, b):
    return pl.pallas_call(
        _mm_body,
        grid=(_N // _TCB,),
        in_specs=[
            pl.BlockSpec((_TCB, _D), lambda i: (i, 0)),
            pl.BlockSpec((_D, 2 * _D), lambda i: (0, 0)),
            pl.BlockSpec((1, 2 * _D), lambda i: (0, 0)),
        ],
        out_specs=[
            pl.BlockSpec((_TCB, _D), lambda i: (i, 0)),
            pl.BlockSpec((_TCB, _D), lambda i: (i, 0)),
        ],
        out_shape=[
            jax.ShapeDtypeStruct((_N, _D), jnp.float32),
            jax.ShapeDtypeStruct((_N, _D), jnp.float32),
        ],
    )(x, w, b)


def _combine_mm_body(agg_ref, degp_ref, yr_ref, g_ref, be_ref, w_ref, b_ref,
                     ol_ref, or_ref):
    inv = 1.0 / jnp.maximum(degp_ref[0] + degp_ref[1], 1.0)
    h = (agg_ref[0] + agg_ref[1]) * inv + yr_ref[...]
    bn_scale = g_ref[...] * (1.0 / (1.0 + _EPS) ** 0.5)
    h = jnp.maximum(h * bn_scale + be_ref[...], 0.0)
    z = (jnp.dot(h, w_ref[...], preferred_element_type=jnp.float32)
         + b_ref[...])
    ol_ref[...] = z[:, :_D]
    or_ref[...] = z[:, _D:]


def _combine_mm(agg, degp, yr, g, be, w, b):
    return pl.pallas_call(
        _combine_mm_body,
        grid=(_N // _TCB,),
        in_specs=[
            pl.BlockSpec((_NC, _TCB, _D), lambda i: (0, i, 0)),
            pl.BlockSpec((_NC, _TCB, _D), lambda i: (0, i, 0)),
            pl.BlockSpec((_TCB, _D), lambda i: (i, 0)),
            pl.BlockSpec((1, _D), lambda i: (0, 0)),
            pl.BlockSpec((1, _D), lambda i: (0, 0)),
            pl.BlockSpec((_D, 2 * _D), lambda i: (0, 0)),
            pl.BlockSpec((1, 2 * _D), lambda i: (0, 0)),
        ],
        out_specs=[
            pl.BlockSpec((_TCB, _D), lambda i: (i, 0)),
            pl.BlockSpec((_TCB, _D), lambda i: (i, 0)),
        ],
        out_shape=[
            jax.ShapeDtypeStruct((_N, _D), jnp.float32),
            jax.ShapeDtypeStruct((_N, _D), jnp.float32),
        ],
    )(agg, degp, yr, g, be, w, b)


def _final_body(agg_ref, degp_ref, yr_ref, o_ref):
    inv = 1.0 / jnp.maximum(degp_ref[0] + degp_ref[1], 1.0)
    o = (agg_ref[0] + agg_ref[1]) * inv + yr_ref[...]
    m = jnp.max(o, axis=-1, keepdims=True)
    lse = jnp.log(jnp.sum(jnp.exp(o - m), axis=-1, keepdims=True)) + m
    o_ref[...] = o - lse


def _final(agg, degp, yr):
    return pl.pallas_call(
        _final_body,
        grid=(_N // _TCB,),
        in_specs=[
            pl.BlockSpec((_NC, _TCB, _D), lambda i: (0, i, 0)),
            pl.BlockSpec((_NC, _TCB, _D), lambda i: (0, i, 0)),
            pl.BlockSpec((_TCB, _D), lambda i: (i, 0)),
        ],
        out_specs=pl.BlockSpec((_TCB, _D), lambda i: (i, 0)),
        out_shape=jax.ShapeDtypeStruct((_N, _D), jnp.float32),
    )(agg, degp, yr)


def kernel(x, edge_index, W1l, b1l, W1r, b1r, g1, be1, W2l, b2l, W2r, b2r):
    srcT = edge_index[0].reshape(_NW, _RNSB, _RSBC, _RCH)
    dstT = edge_index[1].reshape(_NW, _RNSB, _RSBC, _RCH)
    z128 = jnp.zeros((_N, _D), jnp.float32)
    ones_h = jnp.ones((_RCH, _D), jnp.float32)

    Wc1 = jnp.concatenate([W1l, W1r], axis=1)
    bc1 = jnp.concatenate([jnp.zeros_like(b1l), b1l + b1r])[None, :]
    y1l, y1r = _mm(x, Wc1, bc1)

    degp, agg1 = _make_seg_sum(True)(y1l, srcT, dstT, z128, ones_h)

    Wc2 = jnp.concatenate([W2l, W2r], axis=1)
    bc2 = jnp.concatenate([jnp.zeros_like(b2l), b2l + b2r])[None, :]
    y2l, y2r = _combine_mm(agg1, degp, y1r, g1[None, :], be1[None, :],
                           Wc2, bc2)

    agg2 = _make_seg_sum(False)(y2l, srcT, dstT, z128)

    return _final(agg2, degp, y2r)


# speedup vs baseline: 1.2857x; 1.0163x over previous
import functools

import jax
import jax.numpy as jnp
from jax import lax
from jax.experimental import pallas as pl
from jax.experimental.pallas import tpu as pltpu
from jax.experimental.pallas import tpu_sc as plsc

_N = 10000
_E = 320000
_D = 128
_EPS = 1e-5

_NC = 2
_NS = 16
_NW = _NC * _NS
_EPW = _E // _NW
_CH = 125
_NCH = _EPW // _CH
_SBC = 40
_NSB = _NCH // _SBC

_RCH = 100
_RNCH = _EPW // _RCH
_RSBC = 20
_RNSB = _RNCH // _RSBC
_NBUF = 3
_RPT = 624
_RPT_LAST = _N - _RPT * (_NS - 1)
_DGW = 16

_TCB = 2000


def _tile_rows_copy(s, copy):
    @pl.when(s < _NS - 1)
    def _():
        copy(pl.multiple_of(s * _RPT, 8), _RPT)

    @pl.when(s == _NS - 1)
    def _():
        copy(pl.multiple_of(s * _RPT, 8), _RPT_LAST)


def _seg_sum_body(with_deg, *refs):
    if with_deg:
        (y, srcT, dstT, z128, ones_h, deg_out, agg_out,
         acc, src_sb, dst_sb, buf0, buf1, buf2, sem0, sem1, sem2) = refs
    else:
        (y, srcT, dstT, z128, agg_out,
         acc, src_sb, dst_sb, buf0, buf1, buf2, sem0, sem1, sem2) = refs
    c = lax.axis_index("c")
    s = lax.axis_index("s")
    wid = s * _NC + c
    bufs = (buf0, buf1, buf2)
    sems = (sem0, sem1, sem2)
    _tile_rows_copy(s, lambda o, n: pltpu.sync_copy(z128.at[pl.ds(o, n)],
                                                    acc.at[pl.ds(o, n)]))

    if with_deg:
        pltpu.sync_copy(ones_h, buf0)
        plsc.subcore_barrier()

        def deg_superblock(sb, carry):
            pltpu.sync_copy(dstT.at[wid, sb], dst_sb)
            for k in range(_RSBC):
                pltpu.async_copy(buf0, acc.at[dst_sb.at[k]], sem0, add=True)
                if k >= 4:
                    pltpu.make_async_copy(buf0, acc.at[dst_sb.at[0]],
                                          sem0).wait()
            for _ in range(4):
                pltpu.make_async_copy(buf0, acc.at[dst_sb.at[0]],
                                      sem0).wait()
            return carry

        lax.fori_loop(0, _RNSB, deg_superblock, 0)
        plsc.subcore_barrier()
        _tile_rows_copy(s, lambda o, n: pltpu.sync_copy(
            acc.at[pl.ds(o, n)], deg_out.at[c, pl.ds(o, n)]))
        _tile_rows_copy(s, lambda o, n: pltpu.sync_copy(
            z128.at[pl.ds(o, n)], acc.at[pl.ds(o, n)]))

    plsc.subcore_barrier()

    def superblock(sb, carry):
        pltpu.sync_copy(srcT.at[wid, sb], src_sb)
        pltpu.sync_copy(dstT.at[wid, sb], dst_sb)
        for k in range(_NBUF):
            pltpu.async_copy(y.at[src_sb.at[k]], bufs[k], sems[k])
        for k in range(_RSBC):
            b = k % _NBUF
            pltpu.make_async_copy(y.at[src_sb.at[k]], bufs[b],
                                  sems[b]).wait()
            pltpu.sync_copy(bufs[b], acc.at[dst_sb.at[k]], add=True)
            if k + _NBUF < _RSBC:
                pltpu.async_copy(y.at[src_sb.at[k + _NBUF]], bufs[b],
                                 sems[b])
        return carry

    lax.fori_loop(0, _RNSB, superblock, 0)

    plsc.subcore_barrier()
    _tile_rows_copy(s, lambda o, n: pltpu.sync_copy(
        acc.at[pl.ds(o, n)], agg_out.at[c, pl.ds(o, n)]))


def _make_seg_sum(with_deg):
    mesh = plsc.VectorSubcoreMesh(core_axis_name="c", subcore_axis_name="s")
    out = jax.ShapeDtypeStruct((_NC, _N, _D), jnp.float32)
    return pl.kernel(
        functools.partial(_seg_sum_body, with_deg),
        out_type=(out, out) if with_deg else out,
        mesh=mesh,
        scratch_types=[
            pltpu.VMEM_SHARED((_N, _D), jnp.float32),
            pltpu.VMEM((_RSBC, _RCH), jnp.int32),
            pltpu.VMEM((_RSBC, _RCH), jnp.int32),
            pltpu.VMEM((_RCH, _D), jnp.float32),
            pltpu.VMEM((_RCH, _D), jnp.float32),
            pltpu.VMEM((_RCH, _D), jnp.float32),
            pltpu.SemaphoreType.DMA,
            pltpu.SemaphoreType.DMA,
            pltpu.SemaphoreType.DMA,
        ],
    )


def _mm_body(x_ref, w_ref, b_ref, ol_ref, or_ref):
    z = (jnp.dot(x_ref[...], w_ref[...], preferred_element_type=jnp.float32)
         + b_ref[...])
    ol_ref[...] = z[:, :_D]
    or_ref[...] = z[:, _D:]


def _mm(x, w, b):
    return pl.pallas_call(
        _mm_body,
        grid=(_N // _TCB,),
        in_specs=[
            pl.BlockSpec((_TCB, _D), lambda i: (i, 0)),
            pl.BlockSpec((_D, 2 * _D), lambda i: (0, 0)),
            pl.BlockSpec((1, 2 * _D), lambda i: (0, 0)),
        ],
        out_specs=[
            pl.BlockSpec((_TCB, _D), lambda i: (i, 0)),
            pl.BlockSpec((_TCB, _D), lambda i: (i, 0)),
        ],
        out_shape=[
            jax.ShapeDtypeStruct((_N, _D), jnp.float32),
            jax.ShapeDtypeStruct((_N, _D), jnp.float32),
        ],
    )(x, w, b)


def _combine_mm_body(agg_ref, degp_ref, yr_ref, g_ref, be_ref, w_ref, b_ref,
                     ol_ref, or_ref):
    inv = 1.0 / jnp.maximum(degp_ref[0] + degp_ref[1], 1.0)
    h = (agg_ref[0] + agg_ref[1]) * inv + yr_ref[...]
    bn_scale = g_ref[...] * (1.0 / (1.0 + _EPS) ** 0.5)
    h = jnp.maximum(h * bn_scale + be_ref[...], 0.0)
    z = (jnp.dot(h, w_ref[...], preferred_element_type=jnp.float32)
         + b_ref[...])
    ol_ref[...] = z[:, :_D]
    or_ref[...] = z[:, _D:]


def _combine_mm(agg, degp, yr, g, be, w, b):
    return pl.pallas_call(
        _combine_mm_body,
        grid=(_N // _TCB,),
        in_specs=[
            pl.BlockSpec((_NC, _TCB, _D), lambda i: (0, i, 0)),
            pl.BlockSpec((_NC, _TCB, _D), lambda i: (0, i, 0)),
            pl.BlockSpec((_TCB, _D), lambda i: (i, 0)),
            pl.BlockSpec((1, _D), lambda i: (0, 0)),
            pl.BlockSpec((1, _D), lambda i: (0, 0)),
            pl.BlockSpec((_D, 2 * _D), lambda i: (0, 0)),
            pl.BlockSpec((1, 2 * _D), lambda i: (0, 0)),
        ],
        out_specs=[
            pl.BlockSpec((_TCB, _D), lambda i: (i, 0)),
            pl.BlockSpec((_TCB, _D), lambda i: (i, 0)),
        ],
        out_shape=[
            jax.ShapeDtypeStruct((_N, _D), jnp.float32),
            jax.ShapeDtypeStruct((_N, _D), jnp.float32),
        ],
    )(agg, degp, yr, g, be, w, b)


def _final_body(agg_ref, degp_ref, yr_ref, o_ref):
    inv = 1.0 / jnp.maximum(degp_ref[0] + degp_ref[1], 1.0)
    o = (agg_ref[0] + agg_ref[1]) * inv + yr_ref[...]
    m = jnp.max(o, axis=-1, keepdims=True)
    lse = jnp.log(jnp.sum(jnp.exp(o - m), axis=-1, keepdims=True)) + m
    o_ref[...] = o - lse


def _final(agg, degp, yr):
    return pl.pallas_call(
        _final_body,
        grid=(_N // _TCB,),
        in_specs=[
            pl.BlockSpec((_NC, _TCB, _D), lambda i: (0, i, 0)),
            pl.BlockSpec((_NC, _TCB, _D), lambda i: (0, i, 0)),
            pl.BlockSpec((_TCB, _D), lambda i: (i, 0)),
        ],
        out_specs=pl.BlockSpec((_TCB, _D), lambda i: (i, 0)),
        out_shape=jax.ShapeDtypeStruct((_N, _D), jnp.float32),
    )(agg, degp, yr)


def kernel(x, edge_index, W1l, b1l, W1r, b1r, g1, be1, W2l, b2l, W2r, b2r):
    srcT = edge_index[0].reshape(_NW, _RNSB, _RSBC, _RCH)
    dstT = edge_index[1].reshape(_NW, _RNSB, _RSBC, _RCH)
    z128 = jnp.zeros((_N, _D), jnp.float32)
    ones_h = jnp.ones((_RCH, _D), jnp.float32)

    Wc1 = jnp.concatenate([W1l, W1r], axis=1)
    bc1 = jnp.concatenate([jnp.zeros_like(b1l), b1l + b1r])[None, :]
    y1l, y1r = _mm(x, Wc1, bc1)

    degp, agg1 = _make_seg_sum(True)(y1l, srcT, dstT, z128, ones_h)

    Wc2 = jnp.concatenate([W2l, W2r], axis=1)
    bc2 = jnp.concatenate([jnp.zeros_like(b2l), b2l + b2r])[None, :]
    y2l, y2r = _combine_mm(agg1, degp, y1r, g1[None, :], be1[None, :],
                           Wc2, bc2)

    agg2 = _make_seg_sum(False)(y2l, srcT, dstT, z128)

    return _final(agg2, degp, y2r)
